# Initial kernel scaffold; baseline (speedup 1.0000x reference)
import jax
import jax.numpy as jnp
from jax.experimental import pallas as pl


def kernel(cate_list, mask, edge_src, edge_dst, emb, W, a_src, a_dst, ln1_g, ln1_b, W1, b1, W2, b2, ln2_g, ln2_b):
    raise NotImplementedError("write your pallas kernel here")


# speedup vs baseline: 1.0045x; 1.0045x over previous
import functools

import jax
import jax.numpy as jnp
from jax import lax
from jax.experimental import pallas as pl
from jax.experimental.pallas import tpu as pltpu
from jax.experimental.pallas import tpu_sc as plsc

N = 10000
D = 64
E = 320000
FF = 256
HOP = 4
ALPHA = 0.15
LAYERS = 2
B = 1024
S = 50

_info = plsc.get_sparse_core_info()
NC, NS, L = _info.num_cores, _info.num_subcores, _info.num_lanes
NW = NC * NS


def _make_gather_mask(BS, CH):
    per_w = BS // NW
    n_chunks = per_w // CH
    mesh = plsc.VectorSubcoreMesh(core_axis_name="c", subcore_axis_name="s")

    @functools.partial(
        pl.kernel,
        mesh=mesh,
        compiler_params=pltpu.CompilerParams(use_tc_tiling_on_sc=False),
        out_type=jax.ShapeDtypeStruct((BS, D), jnp.float32),
        scratch_types=[
            pltpu.VMEM((CH,), jnp.int32),
            pltpu.VMEM((CH,), jnp.float32),
            pltpu.VMEM((CH, D), jnp.float32),
            pltpu.SemaphoreType.DMA,
        ],
    )
    def k(table_hbm, idx_hbm, mask_hbm, out_hbm, idx_v, mask_v, rows_v, sem):
        wid = lax.axis_index("s") * NC + lax.axis_index("c")
        base = wid * per_w

        def chunk(j, _):
            off = base + j * CH
            pltpu.sync_copy(idx_hbm.at[pl.ds(off, CH)], idx_v)
            pltpu.sync_copy(mask_hbm.at[pl.ds(off, CH)], mask_v)
            pltpu.async_copy(table_hbm.at[idx_v], rows_v, sem).wait()

            def group(g, _):
                mv = mask_v[pl.ds(g * L, L)]
                for lane in range(L):
                    r = g * L + lane
                    bc = jnp.full((L,), mv[lane], jnp.float32)
                    for c in range(D // L):
                        rows_v[r, pl.ds(c * L, L)] = rows_v[r, pl.ds(c * L, L)] * bc
                return 0

            lax.fori_loop(0, CH // L, group, 0)
            pltpu.sync_copy(rows_v, out_hbm.at[pl.ds(off, CH)])
            return 0

        lax.fori_loop(0, n_chunks, chunk, 0)

    return k


def _layernorm(x, g, b):
    mu = jnp.mean(x, axis=-1, keepdims=True)
    var = jnp.var(x, axis=-1, keepdims=True)
    return (x - mu) / jnp.sqrt(var + 1e-5) * g + b


def _magna_layer(h, es, ed, W, a_s, a_d, g1, be1, W1, b1, W2, b2, g2, be2):
    hs = h @ W
    score = jax.nn.leaky_relu(hs[es] @ a_s + hs[ed] @ a_d, negative_slope=0.2)
    smax = jax.ops.segment_max(score, ed, num_segments=N)
    ex = jnp.exp(score - smax[ed])
    denom = jax.ops.segment_sum(ex, ed, num_segments=N)
    attn = ex / (denom[ed] + 1e-16)
    z0 = hs
    z = hs
    for _ in range(HOP):
        agg = jax.ops.segment_sum(attn[:, None] * z[es], ed, num_segments=N)
        z = (1.0 - ALPHA) * agg + ALPHA * z0
    h1 = _layernorm(h + z, g1, be1)
    ff = jax.nn.relu(h1 @ W1 + b1) @ W2 + b2
    return _layernorm(h1 + ff, g2, be2)


def kernel(cate_list, mask, edge_src, edge_dst, emb, W, a_src, a_dst,
           ln1_g, ln1_b, W1, b1, W2, b2, ln2_g, ln2_b):
    h = emb
    for l in range(LAYERS):
        h = _magna_layer(h, edge_src, edge_dst, W[l], a_src[l], a_dst[l],
                         ln1_g[l], ln1_b[l], W1[l], b1[l], W2[l], b2[l],
                         ln2_g[l], ln2_b[l])
    BS = B * S
    idx = jnp.reshape(cate_list, (BS,))
    mv = jnp.reshape(mask, (BS,))
    out = _make_gather_mask(BS, 160)(h, idx, mv)
    return jnp.reshape(out, (B, S, D))


# speedup vs baseline: 7.4856x; 7.4519x over previous
import functools

import jax
import jax.numpy as jnp
from jax import lax
from jax.experimental import pallas as pl
from jax.experimental.pallas import tpu as pltpu
from jax.experimental.pallas import tpu_sc as plsc

N = 10000
D = 64
E = 320000
FF = 256
HOP = 4
ALPHA = 0.15
LAYERS = 2
B = 1024
S = 50

_info = plsc.get_sparse_core_info()
NC, NS, L = _info.num_cores, _info.num_subcores, _info.num_lanes
NW = NC * NS
_SC_PARAMS = pltpu.CompilerParams(
    use_tc_tiling_on_sc=False, needs_layout_passes=False)


def _tc_prologue(h, W, asd):
    BLK = 1000

    def body(h_ref, w_ref, asd_ref, hs_ref, ssd_ref):
        hs = jnp.dot(h_ref[...], w_ref[...], preferred_element_type=jnp.float32)
        hs_ref[...] = hs
        ssd_ref[...] = lax.dot_general(
            hs, asd_ref[...], (((1,), (1,)), ((), ())),
            preferred_element_type=jnp.float32)

    return pl.pallas_call(
        body,
        grid=(N // BLK,),
        in_specs=[
            pl.BlockSpec((BLK, D), lambda i: (i, 0)),
            pl.BlockSpec((D, D), lambda i: (0, 0)),
            pl.BlockSpec((8, D), lambda i: (0, 0)),
        ],
        out_specs=[
            pl.BlockSpec((BLK, D), lambda i: (i, 0)),
            pl.BlockSpec((BLK, 8), lambda i: (i, 0)),
        ],
        out_shape=[
            jax.ShapeDtypeStruct((N, D), jnp.float32),
            jax.ShapeDtypeStruct((N, 8), jnp.float32),
        ],
    )(h, W, asd)


def _make_edge_scores(CH):
    per_w = E // NW
    n_chunks = per_w // CH
    stripe = N // NS
    mesh = plsc.VectorSubcoreMesh(core_axis_name="c", subcore_axis_name="s")

    @functools.partial(
        pl.kernel,
        mesh=mesh,
        compiler_params=_SC_PARAMS,
        out_type=[
            jax.ShapeDtypeStruct((E,), jnp.float32),
            jax.ShapeDtypeStruct((NC, N), jnp.float32),
        ],
        scratch_types=[
            pltpu.VMEM((N,), jnp.float32),
            pltpu.VMEM((N,), jnp.float32),
            pltpu.VMEM((CH,), jnp.int32),
            pltpu.VMEM((CH,), jnp.int32),
            pltpu.VMEM((CH,), jnp.float32),
            pltpu.VMEM((640,), jnp.float32),
            pltpu.VMEM_SHARED((N,), jnp.float32),
        ],
    )
    def k(ssd_hbm, es_hbm, ed_hbm, ex_hbm, den_hbm,
          ss_v, sd_v, es_v, ed_v, ex_v, z_v, den_sh):
        cid = lax.axis_index("c")
        sid = lax.axis_index("s")
        wid = sid * NC + cid
        base = wid * per_w

        def zg(g, _):
            z_v[pl.ds(g * L, L)] = jnp.zeros((L,), jnp.float32)
            return 0
        lax.fori_loop(0, 640 // L, zg, 0)

        @pl.when(sid < NS - 1)
        def _():
            pltpu.sync_copy(z_v, den_sh.at[pl.ds(sid * 640, 640)])

        @pl.when(sid == NS - 1)
        def _():
            pltpu.sync_copy(z_v.at[pl.ds(0, 400)],
                            den_sh.at[pl.ds((NS - 1) * 640, 400)])

        pltpu.sync_copy(ssd_hbm.at[0], ss_v)
        pltpu.sync_copy(ssd_hbm.at[1], sd_v)
        plsc.subcore_barrier()

        def chunk(j, _):
            off = base + j * CH
            pltpu.sync_copy(es_hbm.at[pl.ds(off, CH)], es_v)
            pltpu.sync_copy(ed_hbm.at[pl.ds(off, CH)], ed_v)

            def group(g, _):
                es16 = es_v[pl.ds(g * L, L)]
                ed16 = ed_v[pl.ds(g * L, L)]
                sc = (plsc.load_gather(ss_v, [es16])
                      + plsc.load_gather(sd_v, [ed16]))
                sc = jnp.where(sc >= 0.0, sc, 0.2 * sc)
                ex_v[pl.ds(g * L, L)] = jnp.exp(sc)
                return 0

            lax.fori_loop(0, CH // L, group, 0)
            pltpu.sync_copy(ex_v, ex_hbm.at[pl.ds(off, CH)])
            pltpu.sync_copy(ex_v, den_sh.at[ed_v], add=True)
            return 0

        lax.fori_loop(0, n_chunks, chunk, 0)
        plsc.subcore_barrier()

        @pl.when(sid < NS - 1)
        def _():
            pltpu.sync_copy(den_sh.at[pl.ds(sid * 640, 640)],
                            den_hbm.at[cid, pl.ds(sid * 640, 640)])

        @pl.when(sid == NS - 1)
        def _():
            pltpu.sync_copy(den_sh.at[pl.ds((NS - 1) * 640, 400)],
                            den_hbm.at[cid, pl.ds((NS - 1) * 640, 400)])

    return k


def _make_spmv(CH):
    per_w = E // NW
    n_chunks = per_w // CH
    stripe = N // NS
    mesh = plsc.VectorSubcoreMesh(core_axis_name="c", subcore_axis_name="s")

    @functools.partial(
        pl.kernel,
        mesh=mesh,
        compiler_params=_SC_PARAMS,
        out_type=jax.ShapeDtypeStruct((NC, N, D), jnp.float32),
        scratch_types=[
            pltpu.VMEM((CH,), jnp.int32),
            pltpu.VMEM((CH,), jnp.int32),
            pltpu.VMEM((CH,), jnp.float32),
            pltpu.VMEM((CH, D), jnp.float32),
            pltpu.VMEM((stripe, D), jnp.float32),
            pltpu.VMEM_SHARED((N, D), jnp.float32),
            pltpu.SemaphoreType.DMA,
        ],
    )
    def k(z_hbm, es_hbm, ed_hbm, ex_hbm, out_hbm,
          es_v, ed_v, ex_v, rows_v, zb_v, acc_sh, sem):
        cid = lax.axis_index("c")
        sid = lax.axis_index("s")
        wid = sid * NC + cid
        base = wid * per_w

        def zrow(g, _):
            for c in range(D // L):
                zb_v[g, pl.ds(c * L, L)] = jnp.zeros((L,), jnp.float32)
            return 0
        lax.fori_loop(0, stripe, zrow, 0)
        pltpu.sync_copy(zb_v, acc_sh.at[pl.ds(sid * stripe, stripe)])
        plsc.subcore_barrier()

        def chunk(j, _):
            off = base + j * CH
            pltpu.sync_copy(es_hbm.at[pl.ds(off, CH)], es_v)
            pltpu.sync_copy(ed_hbm.at[pl.ds(off, CH)], ed_v)
            pltpu.sync_copy(ex_hbm.at[pl.ds(off, CH)], ex_v)
            pltpu.async_copy(z_hbm.at[es_v], rows_v, sem).wait()

            def group(g, _):
                exv = ex_v[pl.ds(g * L, L)]
                for lane in range(L):
                    r = g * L + lane
                    bc = jnp.full((L,), exv[lane], jnp.float32)
                    for c in range(D // L):
                        rows_v[r, pl.ds(c * L, L)] = rows_v[r, pl.ds(c * L, L)] * bc
                return 0

            lax.fori_loop(0, CH // L, group, 0)
            pltpu.sync_copy(rows_v, acc_sh.at[ed_v], add=True)
            return 0

        lax.fori_loop(0, n_chunks, chunk, 0)
        plsc.subcore_barrier()
        pltpu.sync_copy(acc_sh.at[pl.ds(sid * stripe, stripe)],
                        out_hbm.at[cid, pl.ds(sid * stripe, stripe)])

    return k


def _tc_combine(parts, den3, z0):
    def body(p_ref, d_ref, z0_ref, o_ref):
        u = p_ref[0] + p_ref[1]
        den2 = d_ref[0] + d_ref[1] + 1e-16
        o_ref[...] = (1.0 - ALPHA) * u / den2 + ALPHA * z0_ref[...]

    return pl.pallas_call(
        body,
        out_shape=jax.ShapeDtypeStruct((N, D), jnp.float32),
    )(parts, den3, z0)


def _tc_epilogue(parts, den, z0, h, g1, be1, W1, b1, W2, b2, g2, be2):
    BLK = 1000

    def ln(x, g, b):
        mu = jnp.mean(x, axis=-1, keepdims=True)
        var = jnp.mean((x - mu) * (x - mu), axis=-1, keepdims=True)
        return (x - mu) / jnp.sqrt(var + 1e-5) * g + b

    def body(p_ref, d_ref, z0_ref, h_ref, g1_ref, b1g_ref, W1_ref, b1_ref,
             W2_ref, b2_ref, g2_ref, b2g_ref, o_ref):
        u = p_ref[0] + p_ref[1]
        den2 = d_ref[0] + d_ref[1] + 1e-16
        z = (1.0 - ALPHA) * u / den2 + ALPHA * z0_ref[...]
        h1 = ln(h_ref[...] + z, g1_ref[...], b1g_ref[...])
        ffa = jnp.maximum(
            jnp.dot(h1, W1_ref[...], preferred_element_type=jnp.float32)
            + b1_ref[...], 0.0)
        ff = jnp.dot(ffa, W2_ref[...], preferred_element_type=jnp.float32) + b2_ref[...]
        o_ref[...] = ln(h1 + ff, g2_ref[...], b2g_ref[...])

    vec = lambda i: (0, 0)
    return pl.pallas_call(
        body,
        grid=(N // BLK,),
        in_specs=[
            pl.BlockSpec((NC, BLK, D), lambda i: (0, i, 0)),
            pl.BlockSpec((NC, BLK, 1), lambda i: (0, i, 0)),
            pl.BlockSpec((BLK, D), lambda i: (i, 0)),
            pl.BlockSpec((BLK, D), lambda i: (i, 0)),
            pl.BlockSpec((1, D), vec),
            pl.BlockSpec((1, D), vec),
            pl.BlockSpec((D, FF), vec),
            pl.BlockSpec((1, FF), vec),
            pl.BlockSpec((FF, D), vec),
            pl.BlockSpec((1, D), vec),
            pl.BlockSpec((1, D), vec),
            pl.BlockSpec((1, D), vec),
        ],
        out_specs=pl.BlockSpec((BLK, D), lambda i: (i, 0)),
        out_shape=jax.ShapeDtypeStruct((N, D), jnp.float32),
    )(parts, den, z0, h, g1, be1, W1, b1, W2, b2, g2, be2)


def _make_gather_mask(BS, CH):
    per_w = BS // NW
    n_chunks = per_w // CH
    mesh = plsc.VectorSubcoreMesh(core_axis_name="c", subcore_axis_name="s")

    @functools.partial(
        pl.kernel,
        mesh=mesh,
        compiler_params=_SC_PARAMS,
        out_type=jax.ShapeDtypeStruct((BS, D), jnp.float32),
        scratch_types=[
            pltpu.VMEM((CH,), jnp.int32),
            pltpu.VMEM((CH,), jnp.float32),
            pltpu.VMEM((CH, D), jnp.float32),
            pltpu.SemaphoreType.DMA,
        ],
    )
    def k(table_hbm, idx_hbm, mask_hbm, out_hbm, idx_v, mask_v, rows_v, sem):
        wid = lax.axis_index("s") * NC + lax.axis_index("c")
        base = wid * per_w

        def chunk(j, _):
            off = base + j * CH
            pltpu.sync_copy(idx_hbm.at[pl.ds(off, CH)], idx_v)
            pltpu.sync_copy(mask_hbm.at[pl.ds(off, CH)], mask_v)
            pltpu.async_copy(table_hbm.at[idx_v], rows_v, sem).wait()

            def group(g, _):
                mv = mask_v[pl.ds(g * L, L)]
                for lane in range(L):
                    r = g * L + lane
                    bc = jnp.full((L,), mv[lane], jnp.float32)
                    for c in range(D // L):
                        rows_v[r, pl.ds(c * L, L)] = rows_v[r, pl.ds(c * L, L)] * bc
                return 0

            lax.fori_loop(0, CH // L, group, 0)
            pltpu.sync_copy(rows_v, out_hbm.at[pl.ds(off, CH)])
            return 0

        lax.fori_loop(0, n_chunks, chunk, 0)

    return k


def kernel(cate_list, mask, edge_src, edge_dst, emb, W, a_src, a_dst,
           ln1_g, ln1_b, W1, b1, W2, b2, ln2_g, ln2_b):
    CH = 400
    edge_scores = _make_edge_scores(CH)
    spmv = _make_spmv(CH)

    h = emb
    for l in range(LAYERS):
        asd = jnp.zeros((8, D), jnp.float32)
        asd = asd.at[0].set(a_src[l]).at[1].set(a_dst[l])
        hs, ssd = _tc_prologue(h, W[l], asd)
        ex, den = edge_scores(jnp.transpose(ssd), edge_src, edge_dst)
        den3 = den[:, :, None]
        z = hs
        for hop in range(HOP):
            parts = spmv(z, edge_src, edge_dst, ex)
            if hop < HOP - 1:
                z = _tc_combine(parts, den3, hs)
        h = _tc_epilogue(parts, den3, hs, h,
                         ln1_g[l][None], ln1_b[l][None], W1[l], b1[l][None],
                         W2[l], b2[l][None], ln2_g[l][None], ln2_b[l][None])

    BS = B * S
    out = _make_gather_mask(BS, 160)(
        h, jnp.reshape(cate_list, (BS,)), jnp.reshape(mask, (BS,)))
    return jnp.reshape(out, (B, S, D))


# speedup vs baseline: 7.6295x; 1.0192x over previous
import functools

import jax
import jax.numpy as jnp
from jax import lax
from jax.experimental import pallas as pl
from jax.experimental.pallas import tpu as pltpu
from jax.experimental.pallas import tpu_sc as plsc

N = 10000
D = 64
E = 320000
FF = 256
HOP = 4
ALPHA = 0.15
LAYERS = 2
B = 1024
S = 50

_info = plsc.get_sparse_core_info()
NC, NS, L = _info.num_cores, _info.num_subcores, _info.num_lanes
NW = NC * NS
_SC_PARAMS = pltpu.CompilerParams(
    use_tc_tiling_on_sc=False, needs_layout_passes=False)


def _tc_prologue(h, W, asd):
    BLK = 1000

    def body(h_ref, w_ref, asd_ref, hs_ref, ssd_ref):
        hs = jnp.dot(h_ref[...], w_ref[...], preferred_element_type=jnp.float32)
        hs_ref[...] = hs
        ssd_ref[...] = lax.dot_general(
            hs, asd_ref[...], (((1,), (1,)), ((), ())),
            preferred_element_type=jnp.float32)

    return pl.pallas_call(
        body,
        grid=(N // BLK,),
        in_specs=[
            pl.BlockSpec((BLK, D), lambda i: (i, 0)),
            pl.BlockSpec((D, D), lambda i: (0, 0)),
            pl.BlockSpec((8, D), lambda i: (0, 0)),
        ],
        out_specs=[
            pl.BlockSpec((BLK, D), lambda i: (i, 0)),
            pl.BlockSpec((BLK, 8), lambda i: (i, 0)),
        ],
        out_shape=[
            jax.ShapeDtypeStruct((N, D), jnp.float32),
            jax.ShapeDtypeStruct((N, 8), jnp.float32),
        ],
    )(h, W, asd)


def _make_edge_scores(CH):
    per_w = E // NW
    n_chunks = per_w // CH
    stripe = N // NS
    mesh = plsc.VectorSubcoreMesh(core_axis_name="c", subcore_axis_name="s")

    @functools.partial(
        pl.kernel,
        mesh=mesh,
        compiler_params=_SC_PARAMS,
        out_type=[
            jax.ShapeDtypeStruct((E,), jnp.float32),
            jax.ShapeDtypeStruct((NC, N), jnp.float32),
        ],
        scratch_types=[
            pltpu.VMEM((N,), jnp.float32),
            pltpu.VMEM((N,), jnp.float32),
            pltpu.VMEM((CH,), jnp.int32),
            pltpu.VMEM((CH,), jnp.int32),
            pltpu.VMEM((CH,), jnp.float32),
            pltpu.VMEM((640,), jnp.float32),
            pltpu.VMEM_SHARED((N,), jnp.float32),
        ],
    )
    def k(ssd_hbm, es_hbm, ed_hbm, ex_hbm, den_hbm,
          ss_v, sd_v, es_v, ed_v, ex_v, z_v, den_sh):
        cid = lax.axis_index("c")
        sid = lax.axis_index("s")
        wid = sid * NC + cid
        base = wid * per_w

        def zg(g, _):
            z_v[pl.ds(g * L, L)] = jnp.zeros((L,), jnp.float32)
            return 0
        lax.fori_loop(0, 640 // L, zg, 0)

        @pl.when(sid < NS - 1)
        def _():
            pltpu.sync_copy(z_v, den_sh.at[pl.ds(sid * 640, 640)])

        @pl.when(sid == NS - 1)
        def _():
            pltpu.sync_copy(z_v.at[pl.ds(0, 400)],
                            den_sh.at[pl.ds((NS - 1) * 640, 400)])

        pltpu.sync_copy(ssd_hbm.at[0], ss_v)
        pltpu.sync_copy(ssd_hbm.at[1], sd_v)
        plsc.subcore_barrier()

        def chunk(j, _):
            off = base + j * CH
            pltpu.sync_copy(es_hbm.at[pl.ds(off, CH)], es_v)
            pltpu.sync_copy(ed_hbm.at[pl.ds(off, CH)], ed_v)

            def group(g, _):
                es16 = es_v[pl.ds(g * L, L)]
                ed16 = ed_v[pl.ds(g * L, L)]
                sc = (plsc.load_gather(ss_v, [es16])
                      + plsc.load_gather(sd_v, [ed16]))
                sc = jnp.where(sc >= 0.0, sc, 0.2 * sc)
                ex_v[pl.ds(g * L, L)] = jnp.exp(sc)
                return 0

            lax.fori_loop(0, CH // L, group, 0)
            pltpu.sync_copy(ex_v, ex_hbm.at[pl.ds(off, CH)])
            pltpu.sync_copy(ex_v, den_sh.at[ed_v], add=True)
            return 0

        lax.fori_loop(0, n_chunks, chunk, 0)
        plsc.subcore_barrier()

        @pl.when(sid < NS - 1)
        def _():
            pltpu.sync_copy(den_sh.at[pl.ds(sid * 640, 640)],
                            den_hbm.at[cid, pl.ds(sid * 640, 640)])

        @pl.when(sid == NS - 1)
        def _():
            pltpu.sync_copy(den_sh.at[pl.ds((NS - 1) * 640, 400)],
                            den_hbm.at[cid, pl.ds((NS - 1) * 640, 400)])

    return k


def _make_spmv(CH):
    per_w = E // NW
    n_chunks = per_w // CH
    stripe = N // NS
    mesh = plsc.VectorSubcoreMesh(core_axis_name="c", subcore_axis_name="s")

    @functools.partial(
        pl.kernel,
        mesh=mesh,
        compiler_params=_SC_PARAMS,
        out_type=jax.ShapeDtypeStruct((NC, N, D), jnp.float32),
        scratch_types=[
            pltpu.VMEM((CH,), jnp.int32),
            pltpu.VMEM((CH,), jnp.int32),
            pltpu.VMEM((CH,), jnp.float32),
            pltpu.VMEM((CH, D), jnp.float32),
            pltpu.VMEM_SHARED((N, D), jnp.float32),
            pltpu.VMEM_SHARED((N, D), jnp.float32),
        ],
    )
    def k(z_hbm, es_hbm, ed_hbm, ex_hbm, out_hbm,
          es_v, ed_v, ex_v, rows_v, z_sh, acc_sh):
        cid = lax.axis_index("c")
        sid = lax.axis_index("s")
        wid = sid * NC + cid
        base = wid * per_w

        pltpu.sync_copy(z_hbm.at[pl.ds(sid * stripe, stripe)],
                        z_sh.at[pl.ds(sid * stripe, stripe)])

        def zrow(g, _):
            for c in range(D // L):
                rows_v[g, pl.ds(c * L, L)] = jnp.zeros((L,), jnp.float32)
            return 0
        lax.fori_loop(0, CH, zrow, 0)
        pltpu.sync_copy(rows_v,
                        acc_sh.at[pl.ds(sid * stripe, CH)])
        pltpu.sync_copy(rows_v.at[pl.ds(0, stripe - CH)],
                        acc_sh.at[pl.ds(sid * stripe + CH, stripe - CH)])
        plsc.subcore_barrier()

        def chunk(j, _):
            off = base + j * CH
            pltpu.sync_copy(es_hbm.at[pl.ds(off, CH)], es_v)
            pltpu.sync_copy(ed_hbm.at[pl.ds(off, CH)], ed_v)
            pltpu.sync_copy(ex_hbm.at[pl.ds(off, CH)], ex_v)
            pltpu.sync_copy(z_sh.at[es_v], rows_v)

            def group(g, _):
                exv = ex_v[pl.ds(g * L, L)]
                for lane in range(L):
                    r = g * L + lane
                    bc = jnp.full((L,), exv[lane], jnp.float32)
                    for c in range(D // L):
                        rows_v[r, pl.ds(c * L, L)] = rows_v[r, pl.ds(c * L, L)] * bc
                return 0

            lax.fori_loop(0, CH // L, group, 0)
            pltpu.sync_copy(rows_v, acc_sh.at[ed_v], add=True)
            return 0

        lax.fori_loop(0, n_chunks, chunk, 0)
        plsc.subcore_barrier()
        pltpu.sync_copy(acc_sh.at[pl.ds(sid * stripe, stripe)],
                        out_hbm.at[cid, pl.ds(sid * stripe, stripe)])

    return k


def _tc_combine(parts, den3, z0):
    def body(p_ref, d_ref, z0_ref, o_ref):
        u = p_ref[0] + p_ref[1]
        den2 = d_ref[0] + d_ref[1] + 1e-16
        o_ref[...] = (1.0 - ALPHA) * u / den2 + ALPHA * z0_ref[...]

    return pl.pallas_call(
        body,
        out_shape=jax.ShapeDtypeStruct((N, D), jnp.float32),
    )(parts, den3, z0)


def _tc_epilogue(parts, den, z0, h, g1, be1, W1, b1, W2, b2, g2, be2):
    BLK = 1000

    def ln(x, g, b):
        mu = jnp.mean(x, axis=-1, keepdims=True)
        var = jnp.mean((x - mu) * (x - mu), axis=-1, keepdims=True)
        return (x - mu) / jnp.sqrt(var + 1e-5) * g + b

    def body(p_ref, d_ref, z0_ref, h_ref, g1_ref, b1g_ref, W1_ref, b1_ref,
             W2_ref, b2_ref, g2_ref, b2g_ref, o_ref):
        u = p_ref[0] + p_ref[1]
        den2 = d_ref[0] + d_ref[1] + 1e-16
        z = (1.0 - ALPHA) * u / den2 + ALPHA * z0_ref[...]
        h1 = ln(h_ref[...] + z, g1_ref[...], b1g_ref[...])
        ffa = jnp.maximum(
            jnp.dot(h1, W1_ref[...], preferred_element_type=jnp.float32)
            + b1_ref[...], 0.0)
        ff = jnp.dot(ffa, W2_ref[...], preferred_element_type=jnp.float32) + b2_ref[...]
        o_ref[...] = ln(h1 + ff, g2_ref[...], b2g_ref[...])

    vec = lambda i: (0, 0)
    return pl.pallas_call(
        body,
        grid=(N // BLK,),
        in_specs=[
            pl.BlockSpec((NC, BLK, D), lambda i: (0, i, 0)),
            pl.BlockSpec((NC, BLK, 1), lambda i: (0, i, 0)),
            pl.BlockSpec((BLK, D), lambda i: (i, 0)),
            pl.BlockSpec((BLK, D), lambda i: (i, 0)),
            pl.BlockSpec((1, D), vec),
            pl.BlockSpec((1, D), vec),
            pl.BlockSpec((D, FF), vec),
            pl.BlockSpec((1, FF), vec),
            pl.BlockSpec((FF, D), vec),
            pl.BlockSpec((1, D), vec),
            pl.BlockSpec((1, D), vec),
            pl.BlockSpec((1, D), vec),
        ],
        out_specs=pl.BlockSpec((BLK, D), lambda i: (i, 0)),
        out_shape=jax.ShapeDtypeStruct((N, D), jnp.float32),
    )(parts, den, z0, h, g1, be1, W1, b1, W2, b2, g2, be2)


def _make_gather_mask(BS, CH):
    per_w = BS // NW
    n_chunks = per_w // CH
    mesh = plsc.VectorSubcoreMesh(core_axis_name="c", subcore_axis_name="s")

    @functools.partial(
        pl.kernel,
        mesh=mesh,
        compiler_params=_SC_PARAMS,
        out_type=jax.ShapeDtypeStruct((BS, D), jnp.float32),
        scratch_types=[
            pltpu.VMEM((CH,), jnp.int32),
            pltpu.VMEM((CH,), jnp.float32),
            pltpu.VMEM((CH, D), jnp.float32),
            pltpu.SemaphoreType.DMA,
        ],
    )
    def k(table_hbm, idx_hbm, mask_hbm, out_hbm, idx_v, mask_v, rows_v, sem):
        wid = lax.axis_index("s") * NC + lax.axis_index("c")
        base = wid * per_w

        def chunk(j, _):
            off = base + j * CH
            pltpu.sync_copy(idx_hbm.at[pl.ds(off, CH)], idx_v)
            pltpu.sync_copy(mask_hbm.at[pl.ds(off, CH)], mask_v)
            pltpu.async_copy(table_hbm.at[idx_v], rows_v, sem).wait()

            def group(g, _):
                mv = mask_v[pl.ds(g * L, L)]
                for lane in range(L):
                    r = g * L + lane
                    bc = jnp.full((L,), mv[lane], jnp.float32)
                    for c in range(D // L):
                        rows_v[r, pl.ds(c * L, L)] = rows_v[r, pl.ds(c * L, L)] * bc
                return 0

            lax.fori_loop(0, CH // L, group, 0)
            pltpu.sync_copy(rows_v, out_hbm.at[pl.ds(off, CH)])
            return 0

        lax.fori_loop(0, n_chunks, chunk, 0)

    return k


def kernel(cate_list, mask, edge_src, edge_dst, emb, W, a_src, a_dst,
           ln1_g, ln1_b, W1, b1, W2, b2, ln2_g, ln2_b):
    CH = 400
    edge_scores = _make_edge_scores(CH)
    spmv = _make_spmv(CH)

    h = emb
    for l in range(LAYERS):
        asd = jnp.zeros((8, D), jnp.float32)
        asd = asd.at[0].set(a_src[l]).at[1].set(a_dst[l])
        hs, ssd = _tc_prologue(h, W[l], asd)
        ex, den = edge_scores(jnp.transpose(ssd), edge_src, edge_dst)
        den3 = den[:, :, None]
        z = hs
        for hop in range(HOP):
            parts = spmv(z, edge_src, edge_dst, ex)
            if hop < HOP - 1:
                z = _tc_combine(parts, den3, hs)
        h = _tc_epilogue(parts, den3, hs, h,
                         ln1_g[l][None], ln1_b[l][None], W1[l], b1[l][None],
                         W2[l], b2[l][None], ln2_g[l][None], ln2_b[l][None])

    BS = B * S
    out = _make_gather_mask(BS, 160)(
        h, jnp.reshape(cate_list, (BS,)), jnp.reshape(mask, (BS,)))
    return jnp.reshape(out, (B, S, D))
